# baseline (speedup 1.0000x reference)
import functools

import jax
import jax.numpy as jnp
from jax import lax
from jax.experimental import pallas as pl
from jax.experimental.pallas import tpu as pltpu
from jax.experimental.pallas import tpu_sc as plsc

_DELTA_VAR = 0.1
_DELTA_DIST = 0.3
_ALPHA = 1.0
_BETA = 1.0

_C = 128
_D = 16
_HW = 128 * 128

_NT = 16
_L = 16


def _tc_body(seg_ref, emb_ref, aff_ref, spmx_ref):
    seg = jnp.reshape(seg_ref[...], (1, _HW))
    emb = jnp.reshape(emb_ref[...], (_D, _HW))
    ids = lax.broadcasted_iota(jnp.int32, (_C, _HW), 0)
    oh = (ids == seg).astype(jnp.float32)
    dn = (((1,), (1,)), ((), ()))
    sp_sums_t = lax.dot_general(emb, oh, dn,
                                preferred_element_type=jnp.float32)
    n_row = lax.dot_general(jnp.ones((1, _HW), jnp.float32), oh, dn,
                            preferred_element_type=jnp.float32)
    inv_n = 1.0 / jnp.maximum(n_row, 1.0)
    means_t = sp_sums_t * inv_n
    dn0 = (((1,), (0,)), ((), ()))
    mean_px = lax.dot_general(means_t, oh, dn0,
                              preferred_element_type=jnp.float32)
    dots = jnp.sum(mean_px * emb, axis=0, keepdims=True)
    t = jnp.clip(1.0 - dots - _DELTA_VAR, 0.0, None)
    seg_t = lax.dot_general(t, oh, dn, preferred_element_type=jnp.float32)
    c_t = (jnp.max(seg) + 1).astype(jnp.float32)
    intra = _BETA * jnp.sum(seg_t * inv_n) / c_t

    a0 = jnp.reshape(aff_ref[0:128, :], (1, _HW))
    a1 = jnp.reshape(aff_ref[128:256, :], (1, _HW))
    hraw = 0.5 * (a0 + a1)
    hmin = jnp.min(hraw)
    hscale = 1.0 / (jnp.max(hraw - hmin) + 1e-6)
    spmx_ref[...] = jnp.concatenate(
        [means_t,
         jnp.full((8, 128), intra, jnp.float32),
         jnp.full((8, 128), hmin, jnp.float32),
         jnp.full((8, 128), hscale, jnp.float32)], axis=0)


def _tc_call(seg2, emb2, aff2):
    return pl.pallas_call(
        _tc_body,
        out_shape=jax.ShapeDtypeStruct((_D + 24, 128), jnp.float32),
    )(seg2, emb2, aff2)


_SPM_INTRA = _D * 128
_SPM_HMIN = (_D + 8) * 128
_SPM_HSCALE = (_D + 16) * 128


def _sc1_body(P, Ppad, Epad,
              aff_hbm, pa_hbm, pb_hbm, pe_hbm,
              out_hbm,
              a0_v, a1_v, pa_v, pb_v, pe1_v, peid_v, vals_v, ones_v,
              sums_v, cnts_v,
              sums_sh, cnts_sh, sem_in, sem_sc):
    cid = lax.axis_index("c")
    sid = lax.axis_index("s")
    pchunk = Ppad // _NT
    echunk = Epad // _NT
    n_pgrp = pchunk // _L
    n_prow = pchunk // 128

    @pl.when(cid == 0)
    def _():
        d_a0 = pltpu.async_copy(aff_hbm.at[pl.ds(0, _HW)], a0_v, sem_in)
        d_a1 = pltpu.async_copy(aff_hbm.at[pl.ds(_HW, _HW)], a1_v, sem_in)
        d_pa = pltpu.async_copy(
            pa_hbm.at[pl.ds(sid * pchunk, pchunk)], pa_v, sem_in)
        d_pb = pltpu.async_copy(
            pb_hbm.at[pl.ds(sid * pchunk, pchunk)], pb_v, sem_in)
        d_pe = pltpu.async_copy(
            pe_hbm.at[pl.ds(sid * pchunk, pchunk)], pe1_v, sem_in)

        def zero_body(j, _):
            off = pl.multiple_of(j * _L, _L)
            sums_v[pl.ds(off, _L)] = jnp.zeros((_L,), jnp.float32)
            return 0
        lax.fori_loop(0, echunk // _L, zero_body, 0)
        pltpu.sync_copy(sums_v, sums_sh.at[pl.ds(sid * echunk, echunk)])
        pltpu.sync_copy(sums_v, cnts_sh.at[pl.ds(sid * echunk, echunk)])

        d_a0.wait(); d_a1.wait(); d_pa.wait(); d_pb.wait(); d_pe.wait()
        base_p = sid * pchunk
        lanes = lax.iota(jnp.int32, _L)

        def pair_body(g, _):
            off = pl.multiple_of(g * _L, _L)
            gidx = base_p + g * _L + lanes
            m = gidx < P
            mi = m.astype(jnp.int32)
            ia = pa_v[pl.ds(off, _L)] * mi
            ib = pb_v[pl.ds(off, _L)] * mi
            ie = pe1_v[pl.ds(off, _L)] * mi
            ha = plsc.load_gather(a0_v, [ia]) + plsc.load_gather(a1_v, [ia])
            hb = plsc.load_gather(a0_v, [ib]) + plsc.load_gather(a1_v, [ib])
            val = 0.25 * (ha + hb)
            q = lax.div(g, 8)
            r = lax.rem(g, 8)
            roff = pl.multiple_of(r * _L, _L)
            peid_v[q, pl.ds(roff, _L)] = ie
            vals_v[q, pl.ds(roff, _L)] = jnp.where(m, val, 0.0)
            ones_v[q, pl.ds(roff, _L)] = jnp.where(m, 1.0, 0.0)
            return 0
        lax.fori_loop(0, n_pgrp, pair_body, 0)

        plsc.subcore_barrier()

        descs = []
        for j in range(n_prow):
            descs.append(pltpu.async_copy(
                vals_v.at[j], sums_sh.at[peid_v.at[j]], sem_sc, add=True))
            descs.append(pltpu.async_copy(
                ones_v.at[j], cnts_sh.at[peid_v.at[j]], sem_sc, add=True))
        for dsc in descs:
            dsc.wait()
        plsc.subcore_barrier()

        base_e = sid * echunk
        d_s = pltpu.async_copy(sums_sh.at[pl.ds(base_e, echunk)], sums_v,
                               sem_in)
        d_c = pltpu.async_copy(cnts_sh.at[pl.ds(base_e, echunk)], cnts_v,
                               sem_in)
        d_s.wait(); d_c.wait()
        pltpu.sync_copy(sums_v, out_hbm.at[pl.ds(base_e, echunk)])
        pltpu.sync_copy(cnts_v, out_hbm.at[pl.ds(Epad + base_e, echunk)])


def _sc1_call(P, Ppad, Epad, aff2, pa, pb, pe):
    pchunk = Ppad // _NT
    echunk = Epad // _NT
    n_prow = pchunk // 128
    mesh = plsc.VectorSubcoreMesh(core_axis_name="c", subcore_axis_name="s",
                                  num_cores=2, num_subcores=_NT)
    kern = pl.kernel(
        functools.partial(_sc1_body, P, Ppad, Epad),
        out_type=jax.ShapeDtypeStruct((2 * Epad,), jnp.float32),
        mesh=mesh,
        compiler_params=pltpu.CompilerParams(needs_layout_passes=False),
        scratch_types=[
            pltpu.VMEM((_HW,), jnp.float32),
            pltpu.VMEM((_HW,), jnp.float32),
            pltpu.VMEM((pchunk,), jnp.int32),
            pltpu.VMEM((pchunk,), jnp.int32),
            pltpu.VMEM((pchunk,), jnp.int32),
            pltpu.VMEM((n_prow, 128), jnp.int32),
            pltpu.VMEM((n_prow, 128), jnp.float32),
            pltpu.VMEM((n_prow, 128), jnp.float32),
            pltpu.VMEM((echunk,), jnp.float32),
            pltpu.VMEM((echunk,), jnp.float32),
            pltpu.VMEM_SHARED((Epad,), jnp.float32),
            pltpu.VMEM_SHARED((Epad,), jnp.float32),
            pltpu.SemaphoreType.DMA,
            pltpu.SemaphoreType.DMA,
        ],
    )
    return kern(aff2, pa, pb, pe)


def _sc2_body(E, Epad,
              sc_hbm, spmx_hbm, edges_hbm,
              out_hbm,
              spm_v, eu_v, ev_v, sums_v, cnts_v, part_v, partall_v, out_v,
              part_sh, sem_in):
    cid = lax.axis_index("c")
    sid = lax.axis_index("s")
    echunk = Epad // _NT
    n_egrp = echunk // _L

    @pl.when(cid == 0)
    def _():
        base_e = sid * echunk
        d_m = pltpu.async_copy(spmx_hbm, spm_v, sem_in)
        d_eu = pltpu.async_copy(
            edges_hbm.at[0, pl.ds(base_e, echunk)], eu_v, sem_in)
        d_ev = pltpu.async_copy(
            edges_hbm.at[1, pl.ds(base_e, echunk)], ev_v, sem_in)
        d_s = pltpu.async_copy(
            sc_hbm.at[pl.ds(base_e, echunk)], sums_v, sem_in)
        d_c = pltpu.async_copy(
            sc_hbm.at[pl.ds(Epad + base_e, echunk)], cnts_v, sem_in)
        d_m.wait(); d_eu.wait(); d_ev.wait(); d_s.wait(); d_c.wait()

        lanes = lax.iota(jnp.int32, _L)
        hminv = spm_v[pl.ds(_SPM_HMIN, _L)]
        hscalev = spm_v[pl.ds(_SPM_HSCALE, _L)]

        def edge_body(g, acc):
            off = pl.multiple_of(g * _L, _L)
            gidx = base_e + g * _L + lanes
            m = gidx < E
            mi = m.astype(jnp.int32)
            u = eu_v[pl.ds(off, _L)] * mi
            v = ev_v[pl.ds(off, _L)] * mi
            s = sums_v[pl.ds(off, _L)]
            c = cnts_v[pl.ds(off, _L)]
            w = (s / jnp.maximum(c, 1.0) - hminv) * hscalev
            dacc = jnp.zeros((_L,), jnp.float32)
            for dd in range(_D):
                mu = plsc.load_gather(spm_v, [u + dd * 128])
                mv = plsc.load_gather(spm_v, [v + dd * 128])
                dacc = dacc + mu * mv
            inter = jnp.clip(_DELTA_DIST - (1.0 - dacc) * w, 0.0, None)
            return acc + jnp.where(m, inter, 0.0)

        acc = lax.fori_loop(0, n_egrp, edge_body, jnp.zeros((_L,), jnp.float32))
        part_v[...] = acc
        pltpu.sync_copy(part_v, part_sh.at[sid])
        plsc.subcore_barrier()

        @pl.when(sid == 0)
        def _():
            pltpu.sync_copy(part_sh, partall_v)
            tot = jnp.zeros((_L,), jnp.float32)
            for t in range(_NT):
                tot = tot + partall_v[t]
            inter_total = lax.reduce(tot, 0.0, lax.add, (0,))
            intra_vec = spm_v[pl.ds(_SPM_INTRA, _L)]
            out_v[...] = jnp.full((_L,), _ALPHA * inter_total * (1.0 / E),
                                  jnp.float32) + intra_vec
            pltpu.sync_copy(out_v, out_hbm)


def _sc2_call(E, Epad, sc1_out, spmx, edges):
    echunk = Epad // _NT
    mesh = plsc.VectorSubcoreMesh(core_axis_name="c", subcore_axis_name="s",
                                  num_cores=2, num_subcores=_NT)
    kern = pl.kernel(
        functools.partial(_sc2_body, E, Epad),
        out_type=jax.ShapeDtypeStruct((_L,), jnp.float32),
        mesh=mesh,
        compiler_params=pltpu.CompilerParams(needs_layout_passes=False),
        scratch_types=[
            pltpu.VMEM(((_D + 24) * 128,), jnp.float32),
            pltpu.VMEM((echunk,), jnp.int32),
            pltpu.VMEM((echunk,), jnp.int32),
            pltpu.VMEM((echunk,), jnp.float32),
            pltpu.VMEM((echunk,), jnp.float32),
            pltpu.VMEM((_L,), jnp.float32),
            pltpu.VMEM((_NT, _L), jnp.float32),
            pltpu.VMEM((_L,), jnp.float32),
            pltpu.VMEM_SHARED((_NT, _L), jnp.float32),
            pltpu.SemaphoreType.DMA,
        ],
    )
    return kern(sc1_out, spmx, edges)


def kernel(embeddings, sp_seg, affs, offs, edges, pair_edge_ids,
           pair_pix_a, pair_pix_b):
    del offs
    seg2 = sp_seg.reshape(128, 128).astype(jnp.int32)
    emb2 = embeddings.reshape(_D * 128, 128).astype(jnp.float32)
    aff2 = affs.reshape(4 * 128, 128).astype(jnp.float32)

    P = pair_pix_a.shape[0]
    E = edges.shape[1]
    Ppad = -(-P // (_NT * 128)) * (_NT * 128)
    Epad = -(-E // (_NT * _L)) * (_NT * _L)

    sc1_out = _sc1_call(P, Ppad, Epad, aff2.reshape(4 * _HW),
                        pair_pix_a.astype(jnp.int32),
                        pair_pix_b.astype(jnp.int32),
                        pair_edge_ids.astype(jnp.int32))
    spmx = _tc_call(seg2, emb2, aff2)
    out_vec = _sc2_call(E, Epad, sc1_out, spmx.reshape((_D + 24) * 128),
                        edges.astype(jnp.int32))
    return out_vec[0]


# speedup vs baseline: 1.0704x; 1.0704x over previous
import functools

import jax
import jax.numpy as jnp
from jax import lax
from jax.experimental import pallas as pl
from jax.experimental.pallas import tpu as pltpu
from jax.experimental.pallas import tpu_sc as plsc

_DELTA_VAR = 0.1
_DELTA_DIST = 0.3
_ALPHA = 1.0
_BETA = 1.0

_C = 128
_D = 16
_HW = 128 * 128

_NT = 16
_L = 16


def _tc_body(seg_ref, emb_ref, aff_ref, spmx_ref):
    seg = jnp.reshape(seg_ref[...], (1, _HW))
    emb = jnp.reshape(emb_ref[...], (_D, _HW))
    ids = lax.broadcasted_iota(jnp.int32, (_C, _HW), 0)
    oh = (ids == seg).astype(jnp.float32)
    dn = (((1,), (1,)), ((), ()))
    sp_sums_t = lax.dot_general(emb, oh, dn,
                                preferred_element_type=jnp.float32)
    n_row = lax.dot_general(jnp.ones((1, _HW), jnp.float32), oh, dn,
                            preferred_element_type=jnp.float32)
    inv_n = 1.0 / jnp.maximum(n_row, 1.0)
    means_t = sp_sums_t * inv_n
    dn0 = (((1,), (0,)), ((), ()))
    mean_px = lax.dot_general(means_t, oh, dn0,
                              preferred_element_type=jnp.float32)
    dots = jnp.sum(mean_px * emb, axis=0, keepdims=True)
    t = jnp.clip(1.0 - dots - _DELTA_VAR, 0.0, None)
    seg_t = lax.dot_general(t, oh, dn, preferred_element_type=jnp.float32)
    c_t = (jnp.max(seg) + 1).astype(jnp.float32)
    intra = _BETA * jnp.sum(seg_t * inv_n) / c_t

    a0 = jnp.reshape(aff_ref[0:128, :], (1, _HW))
    a1 = jnp.reshape(aff_ref[128:256, :], (1, _HW))
    hraw = 0.5 * (a0 + a1)
    hmin = jnp.min(hraw)
    hscale = 1.0 / (jnp.max(hraw - hmin) + 1e-6)
    spmx_ref[...] = jnp.concatenate(
        [means_t,
         jnp.full((8, 128), intra, jnp.float32),
         jnp.full((8, 128), hmin, jnp.float32),
         jnp.full((8, 128), hscale, jnp.float32)], axis=0)


def _tc_call(seg2, emb2, aff2):
    return pl.pallas_call(
        _tc_body,
        out_shape=jax.ShapeDtypeStruct((_D + 24, 128), jnp.float32),
    )(seg2, emb2, aff2)


_SPM_INTRA = _D * 128
_SPM_HMIN = (_D + 8) * 128
_SPM_HSCALE = (_D + 16) * 128


def _sc1_body(P, Ppad, Epad,
              aff_hbm, pa_hbm, pb_hbm, pe_hbm,
              out_hbm,
              hraw_v, a0c_v, a1c_v, hrc_v, pa_v, pb_v, pe1_v, peid_v,
              vals_v, ones_v, sums_v, cnts_v,
              hraw_sh, sums_sh, cnts_sh, sem_in, sem_sc):
    cid = lax.axis_index("c")
    sid = lax.axis_index("s")
    pchunk = Ppad // _NT
    echunk = Epad // _NT
    n_pgrp = pchunk // _L
    n_prow = pchunk // 128
    hchunk = _HW // _NT

    @pl.when(cid == 0)
    def _():
        base_h = sid * hchunk
        d_a0 = pltpu.async_copy(
            aff_hbm.at[pl.ds(base_h, hchunk)], a0c_v, sem_in)
        d_a1 = pltpu.async_copy(
            aff_hbm.at[pl.ds(_HW + base_h, hchunk)], a1c_v, sem_in)
        d_pa = pltpu.async_copy(
            pa_hbm.at[pl.ds(sid * pchunk, pchunk)], pa_v, sem_in)
        d_pb = pltpu.async_copy(
            pb_hbm.at[pl.ds(sid * pchunk, pchunk)], pb_v, sem_in)
        d_pe = pltpu.async_copy(
            pe_hbm.at[pl.ds(sid * pchunk, pchunk)], pe1_v, sem_in)

        def zero_body(j, _):
            off = pl.multiple_of(j * _L, _L)
            sums_v[pl.ds(off, _L)] = jnp.zeros((_L,), jnp.float32)
            return 0
        lax.fori_loop(0, echunk // _L, zero_body, 0)
        pltpu.sync_copy(sums_v, sums_sh.at[pl.ds(sid * echunk, echunk)])
        pltpu.sync_copy(sums_v, cnts_sh.at[pl.ds(sid * echunk, echunk)])

        d_a0.wait(); d_a1.wait()

        def hraw_body(j, _):
            off = pl.multiple_of(j * _L, _L)
            hrc_v[pl.ds(off, _L)] = 0.5 * (a0c_v[pl.ds(off, _L)] +
                                           a1c_v[pl.ds(off, _L)])
            return 0
        lax.fori_loop(0, hchunk // _L, hraw_body, 0)
        pltpu.sync_copy(hrc_v, hraw_sh.at[pl.ds(base_h, hchunk)])
        plsc.subcore_barrier()
        d_h = pltpu.async_copy(hraw_sh, hraw_v, sem_in)

        d_h.wait(); d_pa.wait(); d_pb.wait(); d_pe.wait()
        base_p = sid * pchunk
        lanes = lax.iota(jnp.int32, _L)

        def pair_body(g, _):
            off = pl.multiple_of(g * _L, _L)
            gidx = base_p + g * _L + lanes
            m = gidx < P
            mi = m.astype(jnp.int32)
            ia = pa_v[pl.ds(off, _L)] * mi
            ib = pb_v[pl.ds(off, _L)] * mi
            ie = pe1_v[pl.ds(off, _L)] * mi
            ha = plsc.load_gather(hraw_v, [ia])
            hb = plsc.load_gather(hraw_v, [ib])
            val = 0.5 * (ha + hb)
            q = lax.div(g, 8)
            r = lax.rem(g, 8)
            roff = pl.multiple_of(r * _L, _L)
            peid_v[q, pl.ds(roff, _L)] = ie
            vals_v[q, pl.ds(roff, _L)] = jnp.where(m, val, 0.0)
            ones_v[q, pl.ds(roff, _L)] = jnp.where(m, 1.0, 0.0)
            return 0
        lax.fori_loop(0, n_pgrp, pair_body, 0)

        descs = []
        for j in range(n_prow):
            descs.append(pltpu.async_copy(
                vals_v.at[j], sums_sh.at[peid_v.at[j]], sem_sc, add=True))
            descs.append(pltpu.async_copy(
                ones_v.at[j], cnts_sh.at[peid_v.at[j]], sem_sc, add=True))
        for dsc in descs:
            dsc.wait()
        plsc.subcore_barrier()

        base_e = sid * echunk
        d_s = pltpu.async_copy(sums_sh.at[pl.ds(base_e, echunk)], sums_v,
                               sem_in)
        d_c = pltpu.async_copy(cnts_sh.at[pl.ds(base_e, echunk)], cnts_v,
                               sem_in)
        d_s.wait(); d_c.wait()
        pltpu.sync_copy(sums_v, out_hbm.at[pl.ds(base_e, echunk)])
        pltpu.sync_copy(cnts_v, out_hbm.at[pl.ds(Epad + base_e, echunk)])


def _sc1_call(P, Ppad, Epad, aff2, pa, pb, pe):
    pchunk = Ppad // _NT
    echunk = Epad // _NT
    n_prow = pchunk // 128
    mesh = plsc.VectorSubcoreMesh(core_axis_name="c", subcore_axis_name="s",
                                  num_cores=2, num_subcores=_NT)
    kern = pl.kernel(
        functools.partial(_sc1_body, P, Ppad, Epad),
        out_type=jax.ShapeDtypeStruct((2 * Epad,), jnp.float32),
        mesh=mesh,
        compiler_params=pltpu.CompilerParams(needs_layout_passes=False),
        scratch_types=[
            pltpu.VMEM((_HW,), jnp.float32),
            pltpu.VMEM((_HW // _NT,), jnp.float32),
            pltpu.VMEM((_HW // _NT,), jnp.float32),
            pltpu.VMEM((_HW // _NT,), jnp.float32),
            pltpu.VMEM((pchunk,), jnp.int32),
            pltpu.VMEM((pchunk,), jnp.int32),
            pltpu.VMEM((pchunk,), jnp.int32),
            pltpu.VMEM((n_prow, 128), jnp.int32),
            pltpu.VMEM((n_prow, 128), jnp.float32),
            pltpu.VMEM((n_prow, 128), jnp.float32),
            pltpu.VMEM((echunk,), jnp.float32),
            pltpu.VMEM((echunk,), jnp.float32),
            pltpu.VMEM_SHARED((_HW,), jnp.float32),
            pltpu.VMEM_SHARED((Epad,), jnp.float32),
            pltpu.VMEM_SHARED((Epad,), jnp.float32),
            pltpu.SemaphoreType.DMA,
            pltpu.SemaphoreType.DMA,
        ],
    )
    return kern(aff2, pa, pb, pe)


def _sc2_body(E, Epad,
              sc_hbm, spmx_hbm, edges_hbm,
              out_hbm,
              spm_v, eu_v, ev_v, sums_v, cnts_v, part_v, partall_v, out_v,
              part_sh, sem_in):
    cid = lax.axis_index("c")
    sid = lax.axis_index("s")
    echunk = Epad // _NT
    n_egrp = echunk // _L

    @pl.when(cid == 0)
    def _():
        base_e = sid * echunk
        d_m = pltpu.async_copy(spmx_hbm, spm_v, sem_in)
        d_eu = pltpu.async_copy(
            edges_hbm.at[0, pl.ds(base_e, echunk)], eu_v, sem_in)
        d_ev = pltpu.async_copy(
            edges_hbm.at[1, pl.ds(base_e, echunk)], ev_v, sem_in)
        d_s = pltpu.async_copy(
            sc_hbm.at[pl.ds(base_e, echunk)], sums_v, sem_in)
        d_c = pltpu.async_copy(
            sc_hbm.at[pl.ds(Epad + base_e, echunk)], cnts_v, sem_in)
        d_m.wait(); d_eu.wait(); d_ev.wait(); d_s.wait(); d_c.wait()

        lanes = lax.iota(jnp.int32, _L)
        hminv = spm_v[pl.ds(_SPM_HMIN, _L)]
        hscalev = spm_v[pl.ds(_SPM_HSCALE, _L)]

        def edge_body(g, acc):
            off = pl.multiple_of(g * _L, _L)
            gidx = base_e + g * _L + lanes
            m = gidx < E
            mi = m.astype(jnp.int32)
            u = eu_v[pl.ds(off, _L)] * mi
            v = ev_v[pl.ds(off, _L)] * mi
            s = sums_v[pl.ds(off, _L)]
            c = cnts_v[pl.ds(off, _L)]
            w = (s / jnp.maximum(c, 1.0) - hminv) * hscalev
            dacc = jnp.zeros((_L,), jnp.float32)
            for dd in range(_D):
                mu = plsc.load_gather(spm_v, [u + dd * 128])
                mv = plsc.load_gather(spm_v, [v + dd * 128])
                dacc = dacc + mu * mv
            inter = jnp.clip(_DELTA_DIST - (1.0 - dacc) * w, 0.0, None)
            return acc + jnp.where(m, inter, 0.0)

        acc = lax.fori_loop(0, n_egrp, edge_body, jnp.zeros((_L,), jnp.float32))
        part_v[...] = acc
        pltpu.sync_copy(part_v, part_sh.at[sid])
        plsc.subcore_barrier()

        @pl.when(sid == 0)
        def _():
            pltpu.sync_copy(part_sh, partall_v)
            tot = jnp.zeros((_L,), jnp.float32)
            for t in range(_NT):
                tot = tot + partall_v[t]
            inter_total = lax.reduce(tot, 0.0, lax.add, (0,))
            intra_vec = spm_v[pl.ds(_SPM_INTRA, _L)]
            out_v[...] = jnp.full((_L,), _ALPHA * inter_total * (1.0 / E),
                                  jnp.float32) + intra_vec
            pltpu.sync_copy(out_v, out_hbm)


def _sc2_call(E, Epad, sc1_out, spmx, edges):
    echunk = Epad // _NT
    mesh = plsc.VectorSubcoreMesh(core_axis_name="c", subcore_axis_name="s",
                                  num_cores=2, num_subcores=_NT)
    kern = pl.kernel(
        functools.partial(_sc2_body, E, Epad),
        out_type=jax.ShapeDtypeStruct((_L,), jnp.float32),
        mesh=mesh,
        compiler_params=pltpu.CompilerParams(needs_layout_passes=False),
        scratch_types=[
            pltpu.VMEM(((_D + 24) * 128,), jnp.float32),
            pltpu.VMEM((echunk,), jnp.int32),
            pltpu.VMEM((echunk,), jnp.int32),
            pltpu.VMEM((echunk,), jnp.float32),
            pltpu.VMEM((echunk,), jnp.float32),
            pltpu.VMEM((_L,), jnp.float32),
            pltpu.VMEM((_NT, _L), jnp.float32),
            pltpu.VMEM((_L,), jnp.float32),
            pltpu.VMEM_SHARED((_NT, _L), jnp.float32),
            pltpu.SemaphoreType.DMA,
        ],
    )
    return kern(sc1_out, spmx, edges)


def kernel(embeddings, sp_seg, affs, offs, edges, pair_edge_ids,
           pair_pix_a, pair_pix_b):
    del offs
    seg2 = sp_seg.reshape(128, 128).astype(jnp.int32)
    emb2 = embeddings.reshape(_D * 128, 128).astype(jnp.float32)
    aff2 = affs.reshape(4 * 128, 128).astype(jnp.float32)

    P = pair_pix_a.shape[0]
    E = edges.shape[1]
    Ppad = -(-P // (_NT * 128)) * (_NT * 128)
    Epad = -(-E // (_NT * _L)) * (_NT * _L)

    sc1_out = _sc1_call(P, Ppad, Epad, aff2.reshape(4 * _HW),
                        pair_pix_a.astype(jnp.int32),
                        pair_pix_b.astype(jnp.int32),
                        pair_edge_ids.astype(jnp.int32))
    spmx = _tc_call(seg2, emb2, aff2)
    out_vec = _sc2_call(E, Epad, sc1_out, spmx.reshape((_D + 24) * 128),
                        edges.astype(jnp.int32))
    return out_vec[0]
